# baseline (speedup 1.0000x reference)
import jax
import jax.numpy as jnp
from jax.experimental import pallas as pl

_H = 16
_NH = 8
_HD = 2
_EPS = 1e-5
_L = 9


def _bn(x, g, b):
    mu = jnp.mean(x, axis=0)
    var = jnp.var(x, axis=0)
    return g * (x - mu) / jnp.sqrt(var + _EPS) + b


def _pos_body(pe_ref, w_ref, o_ref):
    o_ref[...] = pe_ref[...] @ w_ref[...]


def kernel(h, pos_enc, edge_index, params):
    p = params
    n = h.shape[0]
    row = edge_index[0]
    col = edge_index[1]
    pos = pl.pallas_call(
        _pos_body,
        out_shape=jax.ShapeDtypeStruct((n, _H), jnp.float32),
    )(pos_enc, p["pos_W"])
    x = p["emb"][h] + pos + p["pos_b"]
    for i in range(_L):
        Wq, bq = p["Wq"][i], p["bq"][i]
        Wk, bk = p["Wk"][i], p["bk"][i]
        Wv, bv = p["Wv"][i], p["bv"][i]
        q = (x @ Wq + bq).reshape(n, _HD, _NH) * (_HD ** -0.5)
        k = (x @ Wk + bk).reshape(n, _HD, _NH)
        v = (x @ Wv + bv).reshape(n, _HD, _NH)
        e = jnp.sum(q[row] * k[col], axis=1)
        m = jax.ops.segment_max(e, row, num_segments=n)
        ex = jnp.exp(e - m[row])
        den = jax.ops.segment_sum(ex, row, num_segments=n)
        a = ex / jnp.maximum(den[row], 1e-9)
        o = jax.ops.segment_sum(a[:, None, :] * v[col], row, num_segments=n).reshape(n, _H)
        o = o @ p["Wo"][i] + p["bo"][i]
        x1 = _bn(x + o, p["bn1_g"][i], p["bn1_b"][i])
        f = jax.nn.relu(x1 @ p["W1"][i] + p["b1"][i]) @ p["W2"][i] + p["b2"][i]
        x = _bn(x1 + f, p["bn2_g"][i], p["bn2_b"][i])
    hg = jnp.mean(x, axis=0, keepdims=True)
    z = jax.nn.relu(hg @ p["cW1"] + p["cb1"])
    z = jax.nn.relu(z @ p["cW2"] + p["cb2"])
    return z @ p["cW3"] + p["cb3"]


# speedup vs baseline: 105.1603x; 105.1603x over previous
import functools

import jax
import jax.numpy as jnp
from jax import lax
from jax.experimental import pallas as pl
from jax.experimental.pallas import tpu as pltpu
from jax.experimental.pallas import tpu_sc as plsc

_N = 50000
_NP = 50048
_E = 800000
_H = 16
_NH = 8
_EPS = 1e-5
_L = 9
_ROWS = 5000
_GRID = _N // _ROWS
_CH = 128
_NCHUNK = _E // _CH
_NW = 32
_TPW = -(-_NCHUNK // _NW)
_ZCH = _NP // _CH
_RPW = _NP // 16
_SCALE = float((_H // _NH) ** -0.5)
_CLAMP = 40.0


def _sc_edge_body(q_hbm, kv_hbm, row_hbm, col_hbm, acc_hbm,
                  acc_sh, row_v, col_v, qbuf, kvbuf, crow, tvec, zvec,
                  sem_i, sem_q, sem_k):
    cid = lax.axis_index("c")
    sid = lax.axis_index("s")
    wid = sid * 2 + cid
    perm = jnp.bitwise_and(lax.iota(jnp.int32, 16) + 8, 15)
    zero16 = jnp.zeros((16,), jnp.float32)
    zvec[...] = zero16

    def _zrow(i, _):
        crow[i, pl.ds(0, 16)] = zero16
        crow[i, pl.ds(8, 16)] = zero16
        return _
    lax.fori_loop(0, _CH, _zrow, None)

    def _zchunk(t, _):
        z = sid * 2 + cid + 32 * t

        @pl.when(z < _ZCH)
        def _():
            pltpu.sync_copy(crow, acc_sh.at[pl.ds(z * _CH, _CH)])
        return _
    lax.fori_loop(0, -(-_ZCH // _NW), _zchunk, None)
    plsc.subcore_barrier()

    def _chunk(t, _):
        z = wid + _NW * t

        @pl.when(z < _NCHUNK)
        def _():
            base = z * _CH
            pltpu.sync_copy(row_hbm.at[pl.ds(base, _CH)], row_v)
            pltpu.sync_copy(col_hbm.at[pl.ds(base, _CH)], col_v)
            cq = pltpu.async_copy(q_hbm.at[row_v], qbuf, sem_q)
            ck = pltpu.async_copy(kv_hbm.at[col_v], kvbuf, sem_k)
            cq.wait()
            ck.wait()

            def _edge(i, _c):
                vq = qbuf[i, :]
                vk = kvbuf[i, pl.ds(0, 16)]
                vv = kvbuf[i, pl.ds(16, 16)]
                tp = vq * vk
                trot = lax.gather(
                    tp, perm[:, None],
                    lax.GatherDimensionNumbers(
                        offset_dims=(), collapsed_slice_dims=(0,),
                        start_index_map=(0,)),
                    slice_sizes=(1,),
                    mode=lax.GatherScatterMode.PROMISE_IN_BOUNDS)
                p = jnp.exp(jnp.minimum(tp + trot, _CLAMP))
                crow[i, pl.ds(0, 16)] = p
                crow[i, pl.ds(8, 16)] = p * vv
                return _c
            lax.fori_loop(0, _CH, _edge, None)
            pltpu.sync_copy(crow, acc_sh.at[row_v], add=True)
        return _
    lax.fori_loop(0, _TPW, _chunk, None)

    plsc.subcore_barrier()
    base = sid * _RPW
    pltpu.sync_copy(acc_sh.at[pl.ds(base, _RPW)],
                    acc_hbm.at[cid, pl.ds(base, _RPW)])


def _sc_edge(q, kv, row, col):
    mesh = plsc.VectorSubcoreMesh(core_axis_name="c", subcore_axis_name="s")
    return pl.kernel(
        _sc_edge_body,
        out_type=jax.ShapeDtypeStruct((2, _NP, 24), jnp.float32),
        mesh=mesh,
        compiler_params=pltpu.CompilerParams(use_tc_tiling_on_sc=False),
        scratch_types=[
            pltpu.VMEM_SHARED((_NP, 24), jnp.float32),
            pltpu.VMEM((_CH,), jnp.int32),
            pltpu.VMEM((_CH,), jnp.int32),
            pltpu.VMEM((_CH, 16), jnp.float32),
            pltpu.VMEM((_CH, 32), jnp.float32),
            pltpu.VMEM((_CH, 24), jnp.float32),
            pltpu.VMEM((16,), jnp.float32),
            pltpu.VMEM((16,), jnp.float32),
            pltpu.SemaphoreType.DMA,
            pltpu.SemaphoreType.DMA,
            pltpu.SemaphoreType.DMA,
        ],
    )(q, kv, row, col)


def _full(shape):
    return pl.BlockSpec(shape, lambda g: (0,) * len(shape))


def _rows(width):
    return pl.BlockSpec((_ROWS, width), lambda g: (g, 0))


def _qkv(x, Wq, bq, Wk, bk, Wv, bv, q_ref, kv_ref):
    q_ref[...] = (x @ Wq + bq) * _SCALE
    k = x @ Wk + bk
    v = x @ Wv + bv
    kv_ref[...] = jnp.concatenate([k, v], axis=1)


def _pre0_body(h_ref, pe_ref, emb_ref, posW_ref, posb_ref,
               Wq_ref, bq_ref, Wk_ref, bk_ref, Wv_ref, bv_ref,
               x_ref, q_ref, kv_ref):
    hv = h_ref[0, 0, :]
    oh = (hv[:, None] == lax.broadcasted_iota(jnp.int32, (_ROWS, 100), 1)
          ).astype(jnp.float32)
    x = oh @ emb_ref[...] + pe_ref[...] @ posW_ref[...] + posb_ref[...]
    x_ref[...] = x
    _qkv(x, Wq_ref[...], bq_ref[...], Wk_ref[...], bk_ref[...],
         Wv_ref[...], bv_ref[...], q_ref, kv_ref)


def _pre0(h3, pos_enc, emb, posW, posb, Wq, bq, Wk, bk, Wv, bv):
    return pl.pallas_call(
        _pre0_body,
        grid=(_GRID,),
        in_specs=[
            pl.BlockSpec((1, 1, _ROWS), lambda g: (g, 0, 0)),
            _rows(8),
            _full((100, _H)), _full((8, _H)), _full((1, _H)),
            _full((_H, _H)), _full((1, _H)),
            _full((_H, _H)), _full((1, _H)),
            _full((_H, _H)), _full((1, _H)),
        ],
        out_specs=[_rows(_H), _rows(_H), _rows(2 * _H)],
        out_shape=[
            jax.ShapeDtypeStruct((_N, _H), jnp.float32),
            jax.ShapeDtypeStruct((_N, _H), jnp.float32),
            jax.ShapeDtypeStruct((_N, 2 * _H), jnp.float32),
        ],
    )(h3, pos_enc, emb, posW, posb, Wq, bq, Wk, bk, Wv, bv)


def _bn_from_stats(y, st, g, b):
    mu = st[0:1, :] * (1.0 / _N)
    var = st[1:2, :] * (1.0 / _N) - mu * mu
    return g * (y - mu) * lax.rsqrt(var + _EPS) + b


def _accum_stats(st_ref, y, gidx):
    bs = jnp.concatenate([jnp.sum(y, axis=0, keepdims=True),
                          jnp.sum(y * y, axis=0, keepdims=True)], axis=0)

    @pl.when(gidx == 0)
    def _():
        st_ref[...] = bs

    @pl.when(gidx != 0)
    def _():
        st_ref[...] = st_ref[...] + bs


def _p1_body(x_ref, accA_ref, accB_ref, Wo_ref, bo_ref, y1_ref, st_ref):
    g = pl.program_id(0)
    acc = accA_ref[0] + accB_ref[0]
    den = jnp.maximum(acc[:, 0:8], 1e-9)
    den16 = jnp.concatenate([den, den], axis=1)
    o = acc[:, 8:24] / den16
    y1 = x_ref[...] + o @ Wo_ref[...] + bo_ref[...]
    y1_ref[...] = y1
    _accum_stats(st_ref, y1, g)


def _p1(x, acc, Wo, bo):
    return pl.pallas_call(
        _p1_body,
        grid=(_GRID,),
        in_specs=[
            _rows(_H),
            pl.BlockSpec((1, _ROWS, 24), lambda g: (0, g, 0)),
            pl.BlockSpec((1, _ROWS, 24), lambda g: (1, g, 0)),
            _full((_H, _H)), _full((1, _H)),
        ],
        out_specs=[_rows(_H), _full((2, _H))],
        out_shape=[
            jax.ShapeDtypeStruct((_N, _H), jnp.float32),
            jax.ShapeDtypeStruct((2, _H), jnp.float32),
        ],
    )(x, acc, acc, Wo, bo)


def _p2_body(y1_ref, st1_ref, g1_ref, b1_ref, W1_ref, bb1_ref,
             W2_ref, bb2_ref, y2_ref, st2_ref):
    g = pl.program_id(0)
    x1 = _bn_from_stats(y1_ref[...], st1_ref[...], g1_ref[...], b1_ref[...])
    f = jax.nn.relu(x1 @ W1_ref[...] + bb1_ref[...]) @ W2_ref[...] + bb2_ref[...]
    y2 = x1 + f
    y2_ref[...] = y2
    _accum_stats(st2_ref, y2, g)


def _p2(y1, st1, g1, b1, W1, bb1, W2, bb2):
    return pl.pallas_call(
        _p2_body,
        grid=(_GRID,),
        in_specs=[
            _rows(_H), _full((2, _H)), _full((1, _H)), _full((1, _H)),
            _full((_H, 2 * _H)), _full((1, 2 * _H)),
            _full((2 * _H, _H)), _full((1, _H)),
        ],
        out_specs=[_rows(_H), _full((2, _H))],
        out_shape=[
            jax.ShapeDtypeStruct((_N, _H), jnp.float32),
            jax.ShapeDtypeStruct((2, _H), jnp.float32),
        ],
    )(y1, st1, g1, b1, W1, bb1, W2, bb2)


def _pre_body(y2_ref, st_ref, g2_ref, b2_ref,
              Wq_ref, bq_ref, Wk_ref, bk_ref, Wv_ref, bv_ref,
              x_ref, q_ref, kv_ref):
    x = _bn_from_stats(y2_ref[...], st_ref[...], g2_ref[...], b2_ref[...])
    x_ref[...] = x
    _qkv(x, Wq_ref[...], bq_ref[...], Wk_ref[...], bk_ref[...],
         Wv_ref[...], bv_ref[...], q_ref, kv_ref)


def _pre(y2, st, g2, b2, Wq, bq, Wk, bk, Wv, bv):
    return pl.pallas_call(
        _pre_body,
        grid=(_GRID,),
        in_specs=[
            _rows(_H), _full((2, _H)), _full((1, _H)), _full((1, _H)),
            _full((_H, _H)), _full((1, _H)),
            _full((_H, _H)), _full((1, _H)),
            _full((_H, _H)), _full((1, _H)),
        ],
        out_specs=[_rows(_H), _rows(_H), _rows(2 * _H)],
        out_shape=[
            jax.ShapeDtypeStruct((_N, _H), jnp.float32),
            jax.ShapeDtypeStruct((_N, _H), jnp.float32),
            jax.ShapeDtypeStruct((_N, 2 * _H), jnp.float32),
        ],
    )(y2, st, g2, b2, Wq, bq, Wk, bk, Wv, bv)


def _fin_body(y2_ref, st_ref, g2_ref, b2_ref, cW1_ref, cb1_ref,
              cW2_ref, cb2_ref, cW3_ref, cb3_ref, out_ref, hsum):
    g = pl.program_id(0)
    x = _bn_from_stats(y2_ref[...], st_ref[...], g2_ref[...], b2_ref[...])
    bs = jnp.sum(x, axis=0, keepdims=True)

    @pl.when(g == 0)
    def _():
        hsum[...] = bs

    @pl.when(g != 0)
    def _():
        hsum[...] = hsum[...] + bs

    @pl.when(g == _GRID - 1)
    def _():
        hg = hsum[...] * (1.0 / _N)
        z = jax.nn.relu(hg @ cW1_ref[...] + cb1_ref[...])
        z = jax.nn.relu(z @ cW2_ref[...] + cb2_ref[...])
        out_ref[...] = z @ cW3_ref[...] + cb3_ref[...]


def _fin(y2, st, g2, b2, cW1, cb1, cW2, cb2, cW3, cb3):
    return pl.pallas_call(
        _fin_body,
        grid=(_GRID,),
        in_specs=[
            _rows(_H), _full((2, _H)), _full((1, _H)), _full((1, _H)),
            _full((_H, 8)), _full((1, 8)),
            _full((8, 4)), _full((1, 4)),
            _full((4, 1)), _full((1, 1)),
        ],
        out_specs=[_full((1, 1))],
        out_shape=[jax.ShapeDtypeStruct((1, 1), jnp.float32)],
        scratch_shapes=[pltpu.VMEM((1, _H), jnp.float32)],
    )(y2, st, g2, b2, cW1, cb1, cW2, cb2, cW3, cb3)[0]


def kernel(h, pos_enc, edge_index, params):
    p = params
    row = edge_index[0]
    col = edge_index[1]
    h3 = h.astype(jnp.int32).reshape(_GRID, 1, _ROWS)

    def r2(b):
        return b.reshape(1, -1)

    x, q, kv = _pre0(h3, pos_enc, p["emb"], p["pos_W"], r2(p["pos_b"]),
                     p["Wq"][0], r2(p["bq"][0]), p["Wk"][0], r2(p["bk"][0]),
                     p["Wv"][0], r2(p["bv"][0]))
    for i in range(_L):
        acc = _sc_edge(q, kv, row, col)
        y1, st1 = _p1(x, acc, p["Wo"][i], r2(p["bo"][i]))
        y2, st2 = _p2(y1, st1, r2(p["bn1_g"][i]), r2(p["bn1_b"][i]),
                      p["W1"][i], r2(p["b1"][i]), p["W2"][i], r2(p["b2"][i]))
        if i + 1 < _L:
            x, q, kv = _pre(y2, st2, r2(p["bn2_g"][i]), r2(p["bn2_b"][i]),
                            p["Wq"][i + 1], r2(p["bq"][i + 1]),
                            p["Wk"][i + 1], r2(p["bk"][i + 1]),
                            p["Wv"][i + 1], r2(p["bv"][i + 1]))
    return _fin(y2, st2, r2(p["bn2_g"][_L - 1]), r2(p["bn2_b"][_L - 1]),
                p["cW1"], r2(p["cb1"]), p["cW2"], r2(p["cb2"]),
                p["cW3"], r2(p["cb3"]))


# speedup vs baseline: 107.6420x; 1.0236x over previous
import functools

import jax
import jax.numpy as jnp
from jax import lax
from jax.experimental import pallas as pl
from jax.experimental.pallas import tpu as pltpu
from jax.experimental.pallas import tpu_sc as plsc

_N = 50000
_NP = 50048
_E = 800000
_H = 16
_NH = 8
_EPS = 1e-5
_L = 9
_ROWS = 5000
_GRID = _N // _ROWS
_CH = 128
_NCHUNK = _E // _CH
_NW = 32
_TPW = -(-_NCHUNK // _NW)
_ZCH = _NP // _CH
_AW = 32
_RPW = _NP // 16
_SCALE = float((_H // _NH) ** -0.5)
_CLAMP = 40.0


def _sc_edge_body(q_hbm, kv_hbm, row_hbm, col_hbm, acc_hbm,
                  acc_sh, row_v, col_v, qbuf, kvbuf, crow, tvec, zvec,
                  sem_i, sem_q, sem_k):
    cid = lax.axis_index("c")
    sid = lax.axis_index("s")
    wid = sid * 2 + cid
    perm = jnp.bitwise_and(lax.iota(jnp.int32, 16) + 8, 15)
    zero16 = jnp.zeros((16,), jnp.float32)
    zvec[...] = zero16

    def _zrow(i, _):
        crow[i, pl.ds(0, 16)] = zero16
        crow[i, pl.ds(16, 16)] = zero16
        return _
    lax.fori_loop(0, _CH, _zrow, None)

    def _zchunk(t, _):
        z = sid + 16 * t

        @pl.when(z < _ZCH)
        def _():
            pltpu.sync_copy(crow, acc_sh.at[pl.ds(z * _CH, _CH)])
        return _
    lax.fori_loop(0, -(-_ZCH // 16), _zchunk, None)
    plsc.subcore_barrier()

    def _chunk(t, _):
        z = wid + _NW * t

        @pl.when(z < _NCHUNK)
        def _():
            base = z * _CH
            pltpu.sync_copy(row_hbm.at[pl.ds(base, _CH)], row_v)
            pltpu.sync_copy(col_hbm.at[pl.ds(base, _CH)], col_v)
            cq = pltpu.async_copy(q_hbm.at[row_v], qbuf, sem_q)
            ck = pltpu.async_copy(kv_hbm.at[col_v], kvbuf, sem_k)
            cq.wait()
            ck.wait()

            def _edge(i, _c):
                vq = qbuf[i, :]
                vk = kvbuf[i, pl.ds(0, 16)]
                vv = kvbuf[i, pl.ds(16, 16)]
                tp = vq * vk
                trot = lax.gather(
                    tp, perm[:, None],
                    lax.GatherDimensionNumbers(
                        offset_dims=(), collapsed_slice_dims=(0,),
                        start_index_map=(0,)),
                    slice_sizes=(1,),
                    mode=lax.GatherScatterMode.PROMISE_IN_BOUNDS)
                p = jnp.exp(jnp.minimum(tp + trot, _CLAMP))
                crow[i, pl.ds(0, 16)] = p
                crow[i, pl.ds(16, 16)] = p * vv
                return _c
            lax.fori_loop(0, _CH, _edge, None)
            pltpu.sync_copy(crow, acc_sh.at[row_v], add=True)
        return _
    lax.fori_loop(0, _TPW, _chunk, None)

    plsc.subcore_barrier()
    base = sid * _RPW
    pltpu.sync_copy(acc_sh.at[pl.ds(base, _RPW)],
                    acc_hbm.at[cid, pl.ds(base, _RPW)])


def _sc_edge(q, kv, row, col):
    mesh = plsc.VectorSubcoreMesh(core_axis_name="c", subcore_axis_name="s")
    return pl.kernel(
        _sc_edge_body,
        out_type=jax.ShapeDtypeStruct((2, _NP, _AW), jnp.float32),
        mesh=mesh,
        compiler_params=pltpu.CompilerParams(use_tc_tiling_on_sc=False),
        scratch_types=[
            pltpu.VMEM_SHARED((_NP, _AW), jnp.float32),
            pltpu.VMEM((_CH,), jnp.int32),
            pltpu.VMEM((_CH,), jnp.int32),
            pltpu.VMEM((_CH, 16), jnp.float32),
            pltpu.VMEM((_CH, 32), jnp.float32),
            pltpu.VMEM((_CH, _AW), jnp.float32),
            pltpu.VMEM((16,), jnp.float32),
            pltpu.VMEM((16,), jnp.float32),
            pltpu.SemaphoreType.DMA,
            pltpu.SemaphoreType.DMA,
            pltpu.SemaphoreType.DMA,
        ],
    )(q, kv, row, col)


def _full(shape):
    return pl.BlockSpec(shape, lambda g: (0,) * len(shape))


def _rows(width):
    return pl.BlockSpec((_ROWS, width), lambda g: (g, 0))


def _qkv(x, Wq, bq, Wk, bk, Wv, bv, q_ref, kv_ref):
    q_ref[...] = (x @ Wq + bq) * _SCALE
    k = x @ Wk + bk
    v = x @ Wv + bv
    kv_ref[...] = jnp.concatenate([k, v], axis=1)


def _pre0_body(h_ref, pe_ref, emb_ref, posW_ref, posb_ref,
               Wq_ref, bq_ref, Wk_ref, bk_ref, Wv_ref, bv_ref,
               x_ref, q_ref, kv_ref):
    hv = h_ref[0, 0, :]
    oh = (hv[:, None] == lax.broadcasted_iota(jnp.int32, (_ROWS, 100), 1)
          ).astype(jnp.float32)
    x = oh @ emb_ref[...] + pe_ref[...] @ posW_ref[...] + posb_ref[...]
    x_ref[...] = x
    _qkv(x, Wq_ref[...], bq_ref[...], Wk_ref[...], bk_ref[...],
         Wv_ref[...], bv_ref[...], q_ref, kv_ref)


def _pre0(h3, pos_enc, emb, posW, posb, Wq, bq, Wk, bk, Wv, bv):
    return pl.pallas_call(
        _pre0_body,
        grid=(_GRID,),
        in_specs=[
            pl.BlockSpec((1, 1, _ROWS), lambda g: (g, 0, 0)),
            _rows(8),
            _full((100, _H)), _full((8, _H)), _full((1, _H)),
            _full((_H, _H)), _full((1, _H)),
            _full((_H, _H)), _full((1, _H)),
            _full((_H, _H)), _full((1, _H)),
        ],
        out_specs=[_rows(_H), _rows(_H), _rows(2 * _H)],
        out_shape=[
            jax.ShapeDtypeStruct((_N, _H), jnp.float32),
            jax.ShapeDtypeStruct((_N, _H), jnp.float32),
            jax.ShapeDtypeStruct((_N, 2 * _H), jnp.float32),
        ],
    )(h3, pos_enc, emb, posW, posb, Wq, bq, Wk, bk, Wv, bv)


def _bn_from_stats(y, st, g, b):
    mu = st[0:1, :] * (1.0 / _N)
    var = st[1:2, :] * (1.0 / _N) - mu * mu
    return g * (y - mu) * lax.rsqrt(var + _EPS) + b


def _accum_stats(st_ref, y, gidx):
    bs = jnp.concatenate([jnp.sum(y, axis=0, keepdims=True),
                          jnp.sum(y * y, axis=0, keepdims=True)], axis=0)

    @pl.when(gidx == 0)
    def _():
        st_ref[...] = bs

    @pl.when(gidx != 0)
    def _():
        st_ref[...] = st_ref[...] + bs


def _p1_body(x_ref, accA_ref, accB_ref, Wo_ref, bo_ref, y1_ref, st_ref):
    g = pl.program_id(0)
    acc = accA_ref[0] + accB_ref[0]
    den = jnp.maximum(acc[:, 0:8], 1e-9)
    den16 = jnp.concatenate([den, den], axis=1)
    o = acc[:, 16:32] / den16
    y1 = x_ref[...] + o @ Wo_ref[...] + bo_ref[...]
    y1_ref[...] = y1
    _accum_stats(st_ref, y1, g)


def _p1(x, acc, Wo, bo):
    return pl.pallas_call(
        _p1_body,
        grid=(_GRID,),
        in_specs=[
            _rows(_H),
            pl.BlockSpec((1, _ROWS, _AW), lambda g: (0, g, 0)),
            pl.BlockSpec((1, _ROWS, _AW), lambda g: (1, g, 0)),
            _full((_H, _H)), _full((1, _H)),
        ],
        out_specs=[_rows(_H), _full((2, _H))],
        out_shape=[
            jax.ShapeDtypeStruct((_N, _H), jnp.float32),
            jax.ShapeDtypeStruct((2, _H), jnp.float32),
        ],
    )(x, acc, acc, Wo, bo)


def _p2_body(y1_ref, st1_ref, g1_ref, b1_ref, W1_ref, bb1_ref,
             W2_ref, bb2_ref, y2_ref, st2_ref):
    g = pl.program_id(0)
    x1 = _bn_from_stats(y1_ref[...], st1_ref[...], g1_ref[...], b1_ref[...])
    f = jax.nn.relu(x1 @ W1_ref[...] + bb1_ref[...]) @ W2_ref[...] + bb2_ref[...]
    y2 = x1 + f
    y2_ref[...] = y2
    _accum_stats(st2_ref, y2, g)


def _p2(y1, st1, g1, b1, W1, bb1, W2, bb2):
    return pl.pallas_call(
        _p2_body,
        grid=(_GRID,),
        in_specs=[
            _rows(_H), _full((2, _H)), _full((1, _H)), _full((1, _H)),
            _full((_H, 2 * _H)), _full((1, 2 * _H)),
            _full((2 * _H, _H)), _full((1, _H)),
        ],
        out_specs=[_rows(_H), _full((2, _H))],
        out_shape=[
            jax.ShapeDtypeStruct((_N, _H), jnp.float32),
            jax.ShapeDtypeStruct((2, _H), jnp.float32),
        ],
    )(y1, st1, g1, b1, W1, bb1, W2, bb2)


def _pre_body(y2_ref, st_ref, g2_ref, b2_ref,
              Wq_ref, bq_ref, Wk_ref, bk_ref, Wv_ref, bv_ref,
              x_ref, q_ref, kv_ref):
    x = _bn_from_stats(y2_ref[...], st_ref[...], g2_ref[...], b2_ref[...])
    x_ref[...] = x
    _qkv(x, Wq_ref[...], bq_ref[...], Wk_ref[...], bk_ref[...],
         Wv_ref[...], bv_ref[...], q_ref, kv_ref)


def _pre(y2, st, g2, b2, Wq, bq, Wk, bk, Wv, bv):
    return pl.pallas_call(
        _pre_body,
        grid=(_GRID,),
        in_specs=[
            _rows(_H), _full((2, _H)), _full((1, _H)), _full((1, _H)),
            _full((_H, _H)), _full((1, _H)),
            _full((_H, _H)), _full((1, _H)),
            _full((_H, _H)), _full((1, _H)),
        ],
        out_specs=[_rows(_H), _rows(_H), _rows(2 * _H)],
        out_shape=[
            jax.ShapeDtypeStruct((_N, _H), jnp.float32),
            jax.ShapeDtypeStruct((_N, _H), jnp.float32),
            jax.ShapeDtypeStruct((_N, 2 * _H), jnp.float32),
        ],
    )(y2, st, g2, b2, Wq, bq, Wk, bk, Wv, bv)


def _fin_body(y2_ref, st_ref, g2_ref, b2_ref, cW1_ref, cb1_ref,
              cW2_ref, cb2_ref, cW3_ref, cb3_ref, out_ref, hsum):
    g = pl.program_id(0)
    x = _bn_from_stats(y2_ref[...], st_ref[...], g2_ref[...], b2_ref[...])
    bs = jnp.sum(x, axis=0, keepdims=True)

    @pl.when(g == 0)
    def _():
        hsum[...] = bs

    @pl.when(g != 0)
    def _():
        hsum[...] = hsum[...] + bs

    @pl.when(g == _GRID - 1)
    def _():
        hg = hsum[...] * (1.0 / _N)
        z = jax.nn.relu(hg @ cW1_ref[...] + cb1_ref[...])
        z = jax.nn.relu(z @ cW2_ref[...] + cb2_ref[...])
        out_ref[...] = z @ cW3_ref[...] + cb3_ref[...]


def _fin(y2, st, g2, b2, cW1, cb1, cW2, cb2, cW3, cb3):
    return pl.pallas_call(
        _fin_body,
        grid=(_GRID,),
        in_specs=[
            _rows(_H), _full((2, _H)), _full((1, _H)), _full((1, _H)),
            _full((_H, 8)), _full((1, 8)),
            _full((8, 4)), _full((1, 4)),
            _full((4, 1)), _full((1, 1)),
        ],
        out_specs=[_full((1, 1))],
        out_shape=[jax.ShapeDtypeStruct((1, 1), jnp.float32)],
        scratch_shapes=[pltpu.VMEM((1, _H), jnp.float32)],
    )(y2, st, g2, b2, cW1, cb1, cW2, cb2, cW3, cb3)[0]


def kernel(h, pos_enc, edge_index, params):
    p = params
    row = edge_index[0]
    col = edge_index[1]
    h3 = h.astype(jnp.int32).reshape(_GRID, 1, _ROWS)

    def r2(b):
        return b.reshape(1, -1)

    x, q, kv = _pre0(h3, pos_enc, p["emb"], p["pos_W"], r2(p["pos_b"]),
                     p["Wq"][0], r2(p["bq"][0]), p["Wk"][0], r2(p["bk"][0]),
                     p["Wv"][0], r2(p["bv"][0]))
    for i in range(_L):
        acc = _sc_edge(q, kv, row, col)
        y1, st1 = _p1(x, acc, p["Wo"][i], r2(p["bo"][i]))
        y2, st2 = _p2(y1, st1, r2(p["bn1_g"][i]), r2(p["bn1_b"][i]),
                      p["W1"][i], r2(p["b1"][i]), p["W2"][i], r2(p["b2"][i]))
        if i + 1 < _L:
            x, q, kv = _pre(y2, st2, r2(p["bn2_g"][i]), r2(p["bn2_b"][i]),
                            p["Wq"][i + 1], r2(p["bq"][i + 1]),
                            p["Wk"][i + 1], r2(p["bk"][i + 1]),
                            p["Wv"][i + 1], r2(p["bv"][i + 1]))
    return _fin(y2, st2, r2(p["bn2_g"][_L - 1]), r2(p["bn2_b"][_L - 1]),
                p["cW1"], r2(p["cb1"]), p["cW2"], r2(p["cb2"]),
                p["cW3"], r2(p["cb3"]))


# speedup vs baseline: 165.9096x; 1.5413x over previous
import functools

import jax
import jax.numpy as jnp
from jax import lax
from jax.experimental import pallas as pl
from jax.experimental.pallas import tpu as pltpu
from jax.experimental.pallas import tpu_sc as plsc

_N = 50000
_NP = 50048
_E = 800000
_H = 16
_NH = 8
_EPS = 1e-5
_L = 9
_ROWS = 5000
_GRID = _N // _ROWS
_CH = 128
_NCHUNK = _E // _CH
_NW = 32
_TPW = -(-_NCHUNK // _NW)
_ZCH = _NP // _CH
_AW = 32
_RPW = _NP // 16
_SCALE = float((_H // _NH) ** -0.5)
_CLAMP = 40.0


def _sc_edge_body(q_hbm, kv_hbm, row_hbm, col_hbm, acc_hbm,
                  acc_sh, row_a, row_b, row_c, row_d,
                  col_a, col_b, col_c, col_d, qbuf, kvbuf, crow,
                  sem_ir, sem_ic, sem_q, sem_k):
    cid = lax.axis_index("c")
    sid = lax.axis_index("s")
    wid = sid * 2 + cid
    perm = jnp.bitwise_and(lax.iota(jnp.int32, 16) + 8, 15)
    zero16 = jnp.zeros((16,), jnp.float32)

    def _zrow(i, _):
        crow[i, pl.ds(0, 16)] = zero16
        crow[i, pl.ds(16, 16)] = zero16
        return _
    lax.fori_loop(0, _CH, _zrow, None)

    def _zchunk(t, _):
        z = sid + 16 * t

        @pl.when(z < _ZCH)
        def _():
            pltpu.sync_copy(crow, acc_sh.at[pl.ds(z * _CH, _CH)])
        return _
    lax.fori_loop(0, -(-_ZCH // 16), _zchunk, None)
    plsc.subcore_barrier()

    rbufs = (row_a, row_b, row_c, row_d)
    cbufs = (col_a, col_b, col_c, col_d)

    def _zof(u):
        return wid + _NW * u

    def _issue_idx(u, s4):
        @pl.when(_zof(u) < _NCHUNK)
        def _():
            base = _zof(u) * _CH
            pltpu.async_copy(row_hbm.at[pl.ds(base, _CH)], rbufs[s4],
                             sem_ir.at[s4])
            pltpu.async_copy(col_hbm.at[pl.ds(base, _CH)], cbufs[s4],
                             sem_ic.at[s4])

    def _wait_idx_issue_gather(u, s4, s2):
        @pl.when(_zof(u) < _NCHUNK)
        def _():
            base = _zof(u) * _CH
            pltpu.make_async_copy(row_hbm.at[pl.ds(base, _CH)], rbufs[s4],
                                  sem_ir.at[s4]).wait()
            pltpu.make_async_copy(col_hbm.at[pl.ds(base, _CH)], cbufs[s4],
                                  sem_ic.at[s4]).wait()
            pltpu.async_copy(q_hbm.at[rbufs[s4]], qbuf.at[s2], sem_q.at[s2])
            pltpu.async_copy(kv_hbm.at[cbufs[s4]], kvbuf.at[s2],
                             sem_k.at[s2])

    def _compute(u, s4, s2):
        @pl.when(_zof(u) < _NCHUNK)
        def _():
            pltpu.make_async_copy(q_hbm.at[rbufs[s4]], qbuf.at[s2],
                                  sem_q.at[s2]).wait()
            pltpu.make_async_copy(kv_hbm.at[cbufs[s4]], kvbuf.at[s2],
                                  sem_k.at[s2]).wait()
            qb = qbuf.at[s2]
            kvb = kvbuf.at[s2]

            def _edge(i4, _c):
                for j in range(4):
                    i = i4 * 4 + j
                    vq = qb[i, :]
                    vk = kvb[i, pl.ds(0, 16)]
                    vv = kvb[i, pl.ds(16, 16)]
                    tp = vq * vk
                    trot = lax.gather(
                        tp, perm[:, None],
                        lax.GatherDimensionNumbers(
                            offset_dims=(), collapsed_slice_dims=(0,),
                            start_index_map=(0,)),
                        slice_sizes=(1,),
                        mode=lax.GatherScatterMode.PROMISE_IN_BOUNDS)
                    p = jnp.exp(jnp.minimum(tp + trot, _CLAMP))
                    crow[i, pl.ds(0, 16)] = p
                    crow[i, pl.ds(16, 16)] = p * vv
                return _c
            lax.fori_loop(0, _CH // 4, _edge, None)
            pltpu.sync_copy(crow, acc_sh.at[rbufs[s4]], add=True)

    _issue_idx(0, 0)
    _issue_idx(1, 1)
    _issue_idx(2, 2)
    _wait_idx_issue_gather(0, 0, 0)

    def _chunk(u4, _):
        u0 = u4 * 4
        for ph in range(4):
            u = u0 + ph
            _wait_idx_issue_gather(u + 1, (ph + 1) % 4, (ph + 1) % 2)
            _compute(u, ph, ph % 2)
            _issue_idx(u + 3, (ph + 3) % 4)
        return _
    lax.fori_loop(0, _TPW // 4, _chunk, None)

    plsc.subcore_barrier()
    base = sid * _RPW
    pltpu.sync_copy(acc_sh.at[pl.ds(base, _RPW)],
                    acc_hbm.at[cid, pl.ds(base, _RPW)])


def _sc_edge(q, kv, row, col):
    mesh = plsc.VectorSubcoreMesh(core_axis_name="c", subcore_axis_name="s")
    return pl.kernel(
        _sc_edge_body,
        out_type=jax.ShapeDtypeStruct((2, _NP, _AW), jnp.float32),
        mesh=mesh,
        compiler_params=pltpu.CompilerParams(use_tc_tiling_on_sc=False),
        scratch_types=[
            pltpu.VMEM_SHARED((_NP, _AW), jnp.float32),
            pltpu.VMEM((_CH,), jnp.int32),
            pltpu.VMEM((_CH,), jnp.int32),
            pltpu.VMEM((_CH,), jnp.int32),
            pltpu.VMEM((_CH,), jnp.int32),
            pltpu.VMEM((_CH,), jnp.int32),
            pltpu.VMEM((_CH,), jnp.int32),
            pltpu.VMEM((_CH,), jnp.int32),
            pltpu.VMEM((_CH,), jnp.int32),
            pltpu.VMEM((2, _CH, 16), jnp.float32),
            pltpu.VMEM((2, _CH, 32), jnp.float32),
            pltpu.VMEM((_CH, _AW), jnp.float32),
            pltpu.SemaphoreType.DMA((4,)),
            pltpu.SemaphoreType.DMA((4,)),
            pltpu.SemaphoreType.DMA((2,)),
            pltpu.SemaphoreType.DMA((2,)),
        ],
    )(q, kv, row, col)


def _full(shape):
    return pl.BlockSpec(shape, lambda g: (0,) * len(shape))


def _rows(width):
    return pl.BlockSpec((_ROWS, width), lambda g: (g, 0))


def _qkv(x, Wq, bq, Wk, bk, Wv, bv, q_ref, kv_ref):
    q_ref[...] = (x @ Wq + bq) * _SCALE
    k = x @ Wk + bk
    v = x @ Wv + bv
    kv_ref[...] = jnp.concatenate([k, v], axis=1)


def _pre0_body(h_ref, pe_ref, emb_ref, posW_ref, posb_ref,
               Wq_ref, bq_ref, Wk_ref, bk_ref, Wv_ref, bv_ref,
               x_ref, q_ref, kv_ref):
    hv = h_ref[0, 0, :]
    oh = (hv[:, None] == lax.broadcasted_iota(jnp.int32, (_ROWS, 100), 1)
          ).astype(jnp.float32)
    x = oh @ emb_ref[...] + pe_ref[...] @ posW_ref[...] + posb_ref[...]
    x_ref[...] = x
    _qkv(x, Wq_ref[...], bq_ref[...], Wk_ref[...], bk_ref[...],
         Wv_ref[...], bv_ref[...], q_ref, kv_ref)


def _pre0(h3, pos_enc, emb, posW, posb, Wq, bq, Wk, bk, Wv, bv):
    return pl.pallas_call(
        _pre0_body,
        grid=(_GRID,),
        in_specs=[
            pl.BlockSpec((1, 1, _ROWS), lambda g: (g, 0, 0)),
            _rows(8),
            _full((100, _H)), _full((8, _H)), _full((1, _H)),
            _full((_H, _H)), _full((1, _H)),
            _full((_H, _H)), _full((1, _H)),
            _full((_H, _H)), _full((1, _H)),
        ],
        out_specs=[_rows(_H), _rows(_H), _rows(2 * _H)],
        out_shape=[
            jax.ShapeDtypeStruct((_N, _H), jnp.float32),
            jax.ShapeDtypeStruct((_N, _H), jnp.float32),
            jax.ShapeDtypeStruct((_N, 2 * _H), jnp.float32),
        ],
    )(h3, pos_enc, emb, posW, posb, Wq, bq, Wk, bk, Wv, bv)


def _bn_from_stats(y, st, g, b):
    mu = st[0:1, :] * (1.0 / _N)
    var = st[1:2, :] * (1.0 / _N) - mu * mu
    return g * (y - mu) * lax.rsqrt(var + _EPS) + b


def _accum_stats(st_ref, y, gidx):
    bs = jnp.concatenate([jnp.sum(y, axis=0, keepdims=True),
                          jnp.sum(y * y, axis=0, keepdims=True)], axis=0)

    @pl.when(gidx == 0)
    def _():
        st_ref[...] = bs

    @pl.when(gidx != 0)
    def _():
        st_ref[...] = st_ref[...] + bs


def _p1_body(x_ref, accA_ref, accB_ref, Wo_ref, bo_ref, y1_ref, st_ref):
    g = pl.program_id(0)
    acc = accA_ref[0] + accB_ref[0]
    den = jnp.maximum(acc[:, 0:8], 1e-9)
    den16 = jnp.concatenate([den, den], axis=1)
    o = acc[:, 16:32] / den16
    y1 = x_ref[...] + o @ Wo_ref[...] + bo_ref[...]
    y1_ref[...] = y1
    _accum_stats(st_ref, y1, g)


def _p1(x, acc, Wo, bo):
    return pl.pallas_call(
        _p1_body,
        grid=(_GRID,),
        in_specs=[
            _rows(_H),
            pl.BlockSpec((1, _ROWS, _AW), lambda g: (0, g, 0)),
            pl.BlockSpec((1, _ROWS, _AW), lambda g: (1, g, 0)),
            _full((_H, _H)), _full((1, _H)),
        ],
        out_specs=[_rows(_H), _full((2, _H))],
        out_shape=[
            jax.ShapeDtypeStruct((_N, _H), jnp.float32),
            jax.ShapeDtypeStruct((2, _H), jnp.float32),
        ],
    )(x, acc, acc, Wo, bo)


def _p2_body(y1_ref, st1_ref, g1_ref, b1_ref, W1_ref, bb1_ref,
             W2_ref, bb2_ref, y2_ref, st2_ref):
    g = pl.program_id(0)
    x1 = _bn_from_stats(y1_ref[...], st1_ref[...], g1_ref[...], b1_ref[...])
    f = jax.nn.relu(x1 @ W1_ref[...] + bb1_ref[...]) @ W2_ref[...] + bb2_ref[...]
    y2 = x1 + f
    y2_ref[...] = y2
    _accum_stats(st2_ref, y2, g)


def _p2(y1, st1, g1, b1, W1, bb1, W2, bb2):
    return pl.pallas_call(
        _p2_body,
        grid=(_GRID,),
        in_specs=[
            _rows(_H), _full((2, _H)), _full((1, _H)), _full((1, _H)),
            _full((_H, 2 * _H)), _full((1, 2 * _H)),
            _full((2 * _H, _H)), _full((1, _H)),
        ],
        out_specs=[_rows(_H), _full((2, _H))],
        out_shape=[
            jax.ShapeDtypeStruct((_N, _H), jnp.float32),
            jax.ShapeDtypeStruct((2, _H), jnp.float32),
        ],
    )(y1, st1, g1, b1, W1, bb1, W2, bb2)


def _pre_body(y2_ref, st_ref, g2_ref, b2_ref,
              Wq_ref, bq_ref, Wk_ref, bk_ref, Wv_ref, bv_ref,
              x_ref, q_ref, kv_ref):
    x = _bn_from_stats(y2_ref[...], st_ref[...], g2_ref[...], b2_ref[...])
    x_ref[...] = x
    _qkv(x, Wq_ref[...], bq_ref[...], Wk_ref[...], bk_ref[...],
         Wv_ref[...], bv_ref[...], q_ref, kv_ref)


def _pre(y2, st, g2, b2, Wq, bq, Wk, bk, Wv, bv):
    return pl.pallas_call(
        _pre_body,
        grid=(_GRID,),
        in_specs=[
            _rows(_H), _full((2, _H)), _full((1, _H)), _full((1, _H)),
            _full((_H, _H)), _full((1, _H)),
            _full((_H, _H)), _full((1, _H)),
            _full((_H, _H)), _full((1, _H)),
        ],
        out_specs=[_rows(_H), _rows(_H), _rows(2 * _H)],
        out_shape=[
            jax.ShapeDtypeStruct((_N, _H), jnp.float32),
            jax.ShapeDtypeStruct((_N, _H), jnp.float32),
            jax.ShapeDtypeStruct((_N, 2 * _H), jnp.float32),
        ],
    )(y2, st, g2, b2, Wq, bq, Wk, bk, Wv, bv)


def _fin_body(y2_ref, st_ref, g2_ref, b2_ref, cW1_ref, cb1_ref,
              cW2_ref, cb2_ref, cW3_ref, cb3_ref, out_ref, hsum):
    g = pl.program_id(0)
    x = _bn_from_stats(y2_ref[...], st_ref[...], g2_ref[...], b2_ref[...])
    bs = jnp.sum(x, axis=0, keepdims=True)

    @pl.when(g == 0)
    def _():
        hsum[...] = bs

    @pl.when(g != 0)
    def _():
        hsum[...] = hsum[...] + bs

    @pl.when(g == _GRID - 1)
    def _():
        hg = hsum[...] * (1.0 / _N)
        z = jax.nn.relu(hg @ cW1_ref[...] + cb1_ref[...])
        z = jax.nn.relu(z @ cW2_ref[...] + cb2_ref[...])
        out_ref[...] = z @ cW3_ref[...] + cb3_ref[...]


def _fin(y2, st, g2, b2, cW1, cb1, cW2, cb2, cW3, cb3):
    return pl.pallas_call(
        _fin_body,
        grid=(_GRID,),
        in_specs=[
            _rows(_H), _full((2, _H)), _full((1, _H)), _full((1, _H)),
            _full((_H, 8)), _full((1, 8)),
            _full((8, 4)), _full((1, 4)),
            _full((4, 1)), _full((1, 1)),
        ],
        out_specs=[_full((1, 1))],
        out_shape=[jax.ShapeDtypeStruct((1, 1), jnp.float32)],
        scratch_shapes=[pltpu.VMEM((1, _H), jnp.float32)],
    )(y2, st, g2, b2, cW1, cb1, cW2, cb2, cW3, cb3)[0]


def kernel(h, pos_enc, edge_index, params):
    p = params
    row = edge_index[0]
    col = edge_index[1]
    h3 = h.astype(jnp.int32).reshape(_GRID, 1, _ROWS)

    def r2(b):
        return b.reshape(1, -1)

    x, q, kv = _pre0(h3, pos_enc, p["emb"], p["pos_W"], r2(p["pos_b"]),
                     p["Wq"][0], r2(p["bq"][0]), p["Wk"][0], r2(p["bk"][0]),
                     p["Wv"][0], r2(p["bv"][0]))
    for i in range(_L):
        acc = _sc_edge(q, kv, row, col)
        y1, st1 = _p1(x, acc, p["Wo"][i], r2(p["bo"][i]))
        y2, st2 = _p2(y1, st1, r2(p["bn1_g"][i]), r2(p["bn1_b"][i]),
                      p["W1"][i], r2(p["b1"][i]), p["W2"][i], r2(p["b2"][i]))
        if i + 1 < _L:
            x, q, kv = _pre(y2, st2, r2(p["bn2_g"][i]), r2(p["bn2_b"][i]),
                            p["Wq"][i + 1], r2(p["bq"][i + 1]),
                            p["Wk"][i + 1], r2(p["bk"][i + 1]),
                            p["Wv"][i + 1], r2(p["bv"][i + 1]))
    return _fin(y2, st2, r2(p["bn2_g"][_L - 1]), r2(p["bn2_b"][_L - 1]),
                p["cW1"], r2(p["cb1"]), p["cW2"], r2(p["cb2"]),
                p["cW3"], r2(p["cb3"]))
